# Initial kernel scaffold; baseline (speedup 1.0000x reference)
import jax
import jax.numpy as jnp
from jax.experimental import pallas as pl


def kernel(query, support):
    raise NotImplementedError("write your pallas kernel here")


# speedup vs baseline: 9.1710x; 9.1710x over previous
import jax
import jax.numpy as jnp
from jax import lax
from jax.experimental import pallas as pl
from jax.experimental.pallas import tpu as pltpu
from jax.experimental.pallas import tpu_sc as plsc

_M = 1024
_D = 16
_N = 100000
_NPAD = 100352
_BN = 1024
_K = 16
_NW = 32
_QPW = _M // _NW
_C = 6272
_NCH = _NPAD // _C
_GPC = _C // 16
_PADVAL = 1e18


def _proxy_body(qm2_ref, s_ref, out_ref):
  s = s_ref[...]
  sn = jnp.sum(s * s, axis=1)
  acc = lax.dot_general(qm2_ref[...], s, (((1,), (1,)), ((), ())),
                        preferred_element_type=jnp.float32)
  out_ref[...] = acc + sn[None, :]


def _compute_proxy(qm2, spad):
  return pl.pallas_call(
      _proxy_body,
      grid=(_NPAD // _BN,),
      in_specs=[
          pl.BlockSpec((_M, _D), lambda i: (0, 0)),
          pl.BlockSpec((_BN, _D), lambda i: (i, 0)),
      ],
      out_specs=pl.BlockSpec((_M, _BN), lambda i: (0, i)),
      out_shape=jax.ShapeDtypeStruct((_M, _NPAD), jnp.float32),
  )(qm2, spad)


def _merge16(bv, bi, cv, ci):
  cs, cis = plsc.sort_key_val(cv, ci)
  cr = lax.rev(cs, (0,))
  cir = lax.rev(cis, (0,))
  take = cr < bv
  nv = jnp.where(take, cr, bv)
  ni = jnp.where(take, cir, bi)
  return plsc.sort_key_val(nv, ni)


def _scan_chunk(buf, chunk_i, carry):
  iota = lax.iota(jnp.int32, 16)
  cbase = chunk_i * _C
  new_carry = []
  for r in range(8):
    bv, bi, thr = carry[3 * r], carry[3 * r + 1], carry[3 * r + 2]

    def group(g, st, r=r):
      bv, bi, thr = st
      vals = buf[r, pl.ds(g * 16, 16)]
      hit = jnp.min(vals) < thr

      def do_merge(bv, bi, thr):
        ci = cbase + g * 16 + iota
        bv, bi = _merge16(bv, bi, vals, ci)
        return bv, bi, bv[15]

      return lax.cond(hit, do_merge, lambda bv, bi, thr: (bv, bi, thr),
                      bv, bi, thr)

    bv, bi, thr = lax.fori_loop(0, _GPC, group, (bv, bi, thr))
    new_carry += [bv, bi, thr]
  return tuple(new_carry)


def _topk_body(proxy, d2_out, idx_out,
               buf0, buf1, res_v, resi_v, sem_a, sem_b):
  c = lax.axis_index("c")
  s = lax.axis_index("s")
  wid = s * 2 + c
  qbase = wid * _QPW

  def octet(o, _):
    rbase = qbase + o * 8

    def slab(cb):
      return proxy.at[pl.ds(rbase, 8), pl.ds(cb, _C)]

    pltpu.async_copy(slab(0), buf0, sem_a)

    init = []
    for _r in range(8):
      init += [jnp.full((16,), jnp.inf, jnp.float32),
               jnp.zeros((16,), jnp.int32), jnp.float32(jnp.inf)]

    def pair(i, carry):
      c0 = 2 * i
      pltpu.async_copy(slab((c0 + 1) * _C), buf1, sem_b)
      pltpu.make_async_copy(slab(c0 * _C), buf0, sem_a).wait()
      carry = _scan_chunk(buf0, c0, carry)

      @pl.when(i < _NCH // 2 - 1)
      def _():
        pltpu.async_copy(slab((c0 + 2) * _C), buf0, sem_a)

      pltpu.make_async_copy(slab((c0 + 1) * _C), buf1, sem_b).wait()
      carry = _scan_chunk(buf1, c0 + 1, carry)
      return carry

    carry = lax.fori_loop(0, _NCH // 2, pair, tuple(init))

    for r in range(8):
      res_v[r] = carry[3 * r]
      resi_v[r] = carry[3 * r + 1]
    pltpu.sync_copy(res_v, d2_out.at[pl.ds(rbase, 8)])
    pltpu.sync_copy(resi_v, idx_out.at[pl.ds(rbase, 8)])
    return 0

  lax.fori_loop(0, _QPW // 8, octet, 0)


def _topk(proxy):
  mesh = plsc.VectorSubcoreMesh(core_axis_name="c", subcore_axis_name="s")
  f = pl.kernel(
      _topk_body,
      out_type=(
          jax.ShapeDtypeStruct((_M, _K), jnp.float32),
          jax.ShapeDtypeStruct((_M, _K), jnp.int32),
      ),
      mesh=mesh,
      scratch_types=[
          pltpu.VMEM((8, _C), jnp.float32),
          pltpu.VMEM((8, _C), jnp.float32),
          pltpu.VMEM((8, _K), jnp.float32),
          pltpu.VMEM((8, _K), jnp.int32),
          pltpu.SemaphoreType.DMA,
          pltpu.SemaphoreType.DMA,
      ],
      compiler_params=pltpu.CompilerParams(needs_layout_passes=False),
  )
  return f(proxy)


def _sqrt_body(bv_ref, q_ref, out_ref):
  q = q_ref[...]
  qn = jnp.sum(q * q, axis=1, keepdims=True)
  out_ref[...] = jnp.sqrt(jnp.maximum(bv_ref[...] + qn, 0.0))


def _sqrt(bv, q):
  return pl.pallas_call(
      _sqrt_body,
      out_shape=jax.ShapeDtypeStruct((_M, _K), jnp.float32),
  )(bv, q)


def kernel(query, support):
  q = query[0]
  s = support[0]
  qm2 = -2.0 * q
  spad = jnp.pad(s, ((0, _NPAD - _N), (0, 0)), constant_values=_PADVAL)
  proxy = _compute_proxy(qm2, spad)
  bv, idx = _topk(proxy)
  values = _sqrt(bv, q)
  return (values.reshape(1, _M, _K), idx.reshape(1, _M, _K))


# speedup vs baseline: 39.1162x; 4.2652x over previous
import jax
import jax.numpy as jnp
from jax import lax
from jax.experimental import pallas as pl
from jax.experimental.pallas import tpu as pltpu
from jax.experimental.pallas import tpu_sc as plsc

_M = 1024
_D = 16
_N = 100000
_NPAD = 100352
_BN = 1024
_K = 16
_NW = 32
_QPW = _M // _NW
_C = 6272
_NCH = _NPAD // _C
_GPC = _C // 16
_PADVAL = 1e18


def _proxy_body(qm2_ref, s_ref, out_ref):
  s = s_ref[...]
  sn = jnp.sum(s * s, axis=1)
  acc = lax.dot_general(qm2_ref[...], s, (((1,), (1,)), ((), ())),
                        preferred_element_type=jnp.float32)
  out_ref[...] = acc + sn[None, :]


def _compute_proxy(qm2, spad):
  return pl.pallas_call(
      _proxy_body,
      grid=(_NPAD // _BN,),
      in_specs=[
          pl.BlockSpec((_M, _D), lambda i: (0, 0)),
          pl.BlockSpec((_BN, _D), lambda i: (i, 0)),
      ],
      out_specs=pl.BlockSpec((_M, _BN), lambda i: (0, i)),
      out_shape=jax.ShapeDtypeStruct((_M, _NPAD), jnp.float32),
  )(qm2, spad)


def _merge16(bv, bi, cv, ci):
  cs, cis = plsc.sort_key_val(cv, ci)
  cr = lax.rev(cs, (0,))
  cir = lax.rev(cis, (0,))
  take = cr < bv
  nv = jnp.where(take, cr, bv)
  ni = jnp.where(take, cir, bi)
  return plsc.sort_key_val(nv, ni)


_BG = 8
_BPC = _GPC // _BG


def _scan_chunk(buf, chunk_i, carry):
  iota = lax.iota(jnp.int32, 16)
  cbase = chunk_i * _C

  def block(b, carry):
    base = b * (_BG * 16)
    out = list(carry)
    for r in range(8):
      bv, bi, thr = carry[3 * r], carry[3 * r + 1], carry[3 * r + 2]
      vs = [buf[r, pl.ds(base + j * 16, 16)] for j in range(_BG)]
      m01 = jnp.minimum(vs[0], vs[1])
      m23 = jnp.minimum(vs[2], vs[3])
      m45 = jnp.minimum(vs[4], vs[5])
      m67 = jnp.minimum(vs[6], vs[7])
      m = jnp.minimum(jnp.minimum(m01, m23), jnp.minimum(m45, m67))
      hit = jnp.min(m) < thr

      def do_block(bv, bi, thr, r=r, base=base):
        def group(j, st):
          bv, bi, thr = st
          vals = buf[r, pl.ds(base + j * 16, 16)]
          ghit = jnp.min(vals) < thr

          def do_merge(bv, bi, thr):
            ci = cbase + base + j * 16 + iota
            bv, bi = _merge16(bv, bi, vals, ci)
            return bv, bi, bv[15]

          return lax.cond(ghit, do_merge,
                          lambda bv, bi, thr: (bv, bi, thr), bv, bi, thr)

        return lax.fori_loop(0, _BG, group, (bv, bi, thr))

      nb = lax.cond(hit, do_block, lambda bv, bi, thr: (bv, bi, thr),
                    bv, bi, thr)
      out[3 * r], out[3 * r + 1], out[3 * r + 2] = nb
    return tuple(out)

  return lax.fori_loop(0, _BPC, block, carry)


def _topk_body(proxy, d2_out, idx_out,
               buf0, buf1, res_v, resi_v, sem_a, sem_b):
  c = lax.axis_index("c")
  s = lax.axis_index("s")
  wid = s * 2 + c
  qbase = wid * _QPW

  def octet(o, _):
    rbase = qbase + o * 8

    def slab(cb):
      return proxy.at[pl.ds(rbase, 8), pl.ds(cb, _C)]

    pltpu.async_copy(slab(0), buf0, sem_a)

    init = []
    for _r in range(8):
      init += [jnp.full((16,), jnp.inf, jnp.float32),
               jnp.zeros((16,), jnp.int32), jnp.float32(jnp.inf)]

    def pair(i, carry):
      c0 = 2 * i
      pltpu.async_copy(slab((c0 + 1) * _C), buf1, sem_b)
      pltpu.make_async_copy(slab(c0 * _C), buf0, sem_a).wait()
      carry = _scan_chunk(buf0, c0, carry)

      @pl.when(i < _NCH // 2 - 1)
      def _():
        pltpu.async_copy(slab((c0 + 2) * _C), buf0, sem_a)

      pltpu.make_async_copy(slab((c0 + 1) * _C), buf1, sem_b).wait()
      carry = _scan_chunk(buf1, c0 + 1, carry)
      return carry

    carry = lax.fori_loop(0, _NCH // 2, pair, tuple(init))

    for r in range(8):
      res_v[r] = carry[3 * r]
      resi_v[r] = carry[3 * r + 1]
    pltpu.sync_copy(res_v, d2_out.at[pl.ds(rbase, 8)])
    pltpu.sync_copy(resi_v, idx_out.at[pl.ds(rbase, 8)])
    return 0

  lax.fori_loop(0, _QPW // 8, octet, 0)


def _topk(proxy):
  mesh = plsc.VectorSubcoreMesh(core_axis_name="c", subcore_axis_name="s")
  f = pl.kernel(
      _topk_body,
      out_type=(
          jax.ShapeDtypeStruct((_M, _K), jnp.float32),
          jax.ShapeDtypeStruct((_M, _K), jnp.int32),
      ),
      mesh=mesh,
      scratch_types=[
          pltpu.VMEM((8, _C), jnp.float32),
          pltpu.VMEM((8, _C), jnp.float32),
          pltpu.VMEM((8, _K), jnp.float32),
          pltpu.VMEM((8, _K), jnp.int32),
          pltpu.SemaphoreType.DMA,
          pltpu.SemaphoreType.DMA,
      ],
      compiler_params=pltpu.CompilerParams(needs_layout_passes=False),
  )
  return f(proxy)


def _sqrt_body(bv_ref, q_ref, out_ref):
  q = q_ref[...]
  qn = jnp.sum(q * q, axis=1, keepdims=True)
  out_ref[...] = jnp.sqrt(jnp.maximum(bv_ref[...] + qn, 0.0))


def _sqrt(bv, q):
  return pl.pallas_call(
      _sqrt_body,
      out_shape=jax.ShapeDtypeStruct((_M, _K), jnp.float32),
  )(bv, q)


def kernel(query, support):
  q = query[0]
  s = support[0]
  qm2 = -2.0 * q
  spad = jnp.pad(s, ((0, _NPAD - _N), (0, 0)), constant_values=_PADVAL)
  proxy = _compute_proxy(qm2, spad)
  bv, idx = _topk(proxy)
  values = _sqrt(bv, q)
  return (values.reshape(1, _M, _K), idx.reshape(1, _M, _K))
